# baseline (speedup 1.0000x reference)
import dataclasses

import jax
import jax.numpy as jnp
from jax.experimental import pallas as pl
from jax.experimental.pallas import tpu as pltpu
from jax.experimental.pallas import tpu_sc as plsc

B = 1024
SEQ = 77
CTX_DIM = 512
N_CTX = 4
KEEP = SEQ - N_CTX

NUM_CORES = 2
NUM_SUBCORES = 16
HALF = B // NUM_CORES
BATCH_PER_W = HALF // NUM_SUBCORES
NBUF = 2

MERGE_BLK = 32


def _sc_body(txt0_hbm, txt1_hbm, ctx_hbm, table_hbm, out0_hbm, out1_hbm,
             idx_v, ctx_v, gbuf0, gbuf1, sidx, cidx,
             gsem0, gsem1, wsem0, wsem1, csem):
    cid = jax.lax.axis_index("c")
    sid = jax.lax.axis_index("s")
    base = sid * BATCH_PER_W

    gbufs = (gbuf0, gbuf1)
    gsems = (gsem0, gsem1)
    wsems = (wsem0, wsem1)

    lane = jax.lax.iota(jnp.int32, 16)
    for c in range(KEEP // 16):
        sidx.at[pl.ds(16 * c, 16)][...] = lane + (N_CTX + 16 * c)
    tail = (KEEP // 16) * 16
    plsc.store_scatter(sidx, [lane + tail], lane + (N_CTX + tail),
                       mask=lane < (KEEP - tail))
    plsc.store_scatter(cidx, [lane], lane, mask=lane < N_CTX)

    pltpu.sync_copy(ctx_hbm, ctx_v)

    def run_half(txt_hbm, out_hbm):
        pltpu.sync_copy(txt_hbm.at[pl.ds(base, BATCH_PER_W)], idx_v)

        def gather_copy(j, b):
            return pltpu.make_async_copy(
                table_hbm.at[idx_v.at[j, pl.ds(0, KEEP)]], gbufs[b], gsems[b])

        def scatter_copy(j, b):
            return pltpu.make_async_copy(
                gbufs[b], out_hbm.at[base + j].at[sidx], wsems[b])

        def ctx_copy(j):
            return pltpu.make_async_copy(
                ctx_v, out_hbm.at[base + j].at[cidx], csem)

        gather_copy(0, 0).start()
        gather_copy(1, 1).start()

        def step(j, b, refire):
            ctx_copy(j).start()
            gather_copy(j, b).wait()
            scatter_copy(j, b).start()
            scatter_copy(j, b).wait()
            if refire:
                gather_copy(j + NBUF, b).start()
            ctx_copy(j).wait()

        @pl.loop(0, BATCH_PER_W // NBUF - 1)
        def _(g):
            for b in range(NBUF):
                step(NBUF * g + b, b, refire=True)

        for b in range(NBUF):
            step(BATCH_PER_W - NBUF + b, b, refire=False)

    @pl.when(cid == 0)
    def _():
        run_half(txt0_hbm, out0_hbm)

    @pl.when(cid == 1)
    def _():
        run_half(txt1_hbm, out1_hbm)


def _merge_body(h0_ref, h1_ref, out_ref):
    out_ref[:, 0] = h0_ref[...]
    out_ref[:, 1] = h1_ref[...]


@jax.jit
def _prompt_gather(text, ctx, table):
    cp = pltpu.CompilerParams()
    if "needs_layout_passes" in pltpu.CompilerParams.__dataclass_fields__:
        cp = dataclasses.replace(cp, needs_layout_passes=False)
    sc_kernel = pl.kernel(
        _sc_body,
        compiler_params=cp,
        out_type=[
            jax.ShapeDtypeStruct((HALF, SEQ, CTX_DIM), jnp.float32),
            jax.ShapeDtypeStruct((HALF, SEQ, CTX_DIM), jnp.float32),
        ],
        mesh=plsc.VectorSubcoreMesh(core_axis_name="c", subcore_axis_name="s"),
        scratch_types=[
            pltpu.VMEM((BATCH_PER_W, SEQ), jnp.int32),
            pltpu.VMEM((N_CTX, CTX_DIM), jnp.float32),
            pltpu.VMEM((KEEP, CTX_DIM), jnp.float32),
            pltpu.VMEM((KEEP, CTX_DIM), jnp.float32),
            pltpu.VMEM((KEEP,), jnp.int32),
            pltpu.VMEM((N_CTX,), jnp.int32),
            pltpu.SemaphoreType.DMA,
            pltpu.SemaphoreType.DMA,
            pltpu.SemaphoreType.DMA,
            pltpu.SemaphoreType.DMA,
            pltpu.SemaphoreType.DMA,
        ],
    )
    out0, out1 = sc_kernel(text[0::2], text[1::2], ctx, table)

    merged = pl.pallas_call(
        _merge_body,
        grid=(HALF // MERGE_BLK,),
        in_specs=[
            pl.BlockSpec((MERGE_BLK, SEQ, CTX_DIM), lambda i: (i, 0, 0)),
            pl.BlockSpec((MERGE_BLK, SEQ, CTX_DIM), lambda i: (i, 0, 0)),
        ],
        out_specs=pl.BlockSpec((MERGE_BLK, NUM_CORES, SEQ, CTX_DIM),
                               lambda i: (i, 0, 0, 0)),
        out_shape=jax.ShapeDtypeStruct((HALF, NUM_CORES, SEQ, CTX_DIM),
                                       jnp.float32),
    )(out0, out1)
    return merged.reshape(B, SEQ, CTX_DIM)


def kernel(text, ctx, token_embedding):
    return _prompt_gather(text, ctx, token_embedding)


# speedup vs baseline: 1.0283x; 1.0283x over previous
import dataclasses

import jax
import jax.numpy as jnp
from jax.experimental import pallas as pl
from jax.experimental.pallas import tpu as pltpu
from jax.experimental.pallas import tpu_sc as plsc

B = 1024
SEQ = 77
CTX_DIM = 512
N_CTX = 4
KEEP = SEQ - N_CTX

NUM_CORES = 2
NUM_SUBCORES = 16
HALF = B // NUM_CORES
BATCH_PER_W = HALF // NUM_SUBCORES
NBUF = 2

MERGE_BLK = 32


def _sc_body(txt0_hbm, txt1_hbm, ctx_hbm, table_hbm, out0_hbm, out1_hbm,
             idx_v, ctx_v, gbuf0, gbuf1, sidx, cidx,
             gsem0, gsem1, wsem0, wsem1, csem):
    cid = jax.lax.axis_index("c")
    sid = jax.lax.axis_index("s")
    base = sid * BATCH_PER_W

    gbufs = (gbuf0, gbuf1)
    gsems = (gsem0, gsem1)
    wsems = (wsem0, wsem1)

    lane = jax.lax.iota(jnp.int32, 16)
    for c in range(KEEP // 16):
        sidx.at[pl.ds(16 * c, 16)][...] = lane + (N_CTX + 16 * c)
    tail = (KEEP // 16) * 16
    plsc.store_scatter(sidx, [lane + tail], lane + (N_CTX + tail),
                       mask=lane < (KEEP - tail))
    plsc.store_scatter(cidx, [lane], lane, mask=lane < N_CTX)

    pltpu.sync_copy(ctx_hbm, ctx_v)

    def run_half(txt_hbm, out_hbm):
        pltpu.sync_copy(txt_hbm.at[pl.ds(base, BATCH_PER_W)], idx_v)

        def gather_copy(j, b):
            return pltpu.make_async_copy(
                table_hbm.at[idx_v.at[j, pl.ds(0, KEEP)]], gbufs[b], gsems[b])

        def scatter_copy(j, b):
            return pltpu.make_async_copy(
                gbufs[b], out_hbm.at[base + j].at[sidx], wsems[b])

        def ctx_copy(j):
            return pltpu.make_async_copy(
                ctx_v, out_hbm.at[base + j].at[cidx], csem)

        gather_copy(0, 0).start()
        gather_copy(1, 1).start()

        def step(j, b, refire):
            ctx_copy(j).start()
            gather_copy(j, b).wait()
            scatter_copy(j, b).start()
            scatter_copy(j, b).wait()
            if refire:
                gather_copy(j + NBUF, b).start()
            ctx_copy(j).wait()

        @pl.loop(0, BATCH_PER_W // NBUF - 1)
        def _(g):
            for b in range(NBUF):
                step(NBUF * g + b, b, refire=True)

        for b in range(NBUF):
            step(BATCH_PER_W - NBUF + b, b, refire=False)

    @pl.when(cid == 0)
    def _():
        run_half(txt0_hbm, out0_hbm)

    @pl.when(cid == 1)
    def _():
        run_half(txt1_hbm, out1_hbm)


def _merge_body(h0_ref, h1_ref, out_ref):
    for k in range(MERGE_BLK):
        out_ref[2 * k] = h0_ref[k]
        out_ref[2 * k + 1] = h1_ref[k]


@jax.jit
def _prompt_gather(text, ctx, table):
    cp = pltpu.CompilerParams()
    if "needs_layout_passes" in pltpu.CompilerParams.__dataclass_fields__:
        cp = dataclasses.replace(cp, needs_layout_passes=False)
    sc_kernel = pl.kernel(
        _sc_body,
        compiler_params=cp,
        out_type=[
            jax.ShapeDtypeStruct((HALF, SEQ, CTX_DIM), jnp.float32),
            jax.ShapeDtypeStruct((HALF, SEQ, CTX_DIM), jnp.float32),
        ],
        mesh=plsc.VectorSubcoreMesh(core_axis_name="c", subcore_axis_name="s"),
        scratch_types=[
            pltpu.VMEM((BATCH_PER_W, SEQ), jnp.int32),
            pltpu.VMEM((N_CTX, CTX_DIM), jnp.float32),
            pltpu.VMEM((KEEP, CTX_DIM), jnp.float32),
            pltpu.VMEM((KEEP, CTX_DIM), jnp.float32),
            pltpu.VMEM((KEEP,), jnp.int32),
            pltpu.VMEM((N_CTX,), jnp.int32),
            pltpu.SemaphoreType.DMA,
            pltpu.SemaphoreType.DMA,
            pltpu.SemaphoreType.DMA,
            pltpu.SemaphoreType.DMA,
            pltpu.SemaphoreType.DMA,
        ],
    )
    out0, out1 = sc_kernel(text[0::2], text[1::2], ctx, table)

    merged = pl.pallas_call(
        _merge_body,
        grid=(HALF // MERGE_BLK,),
        in_specs=[
            pl.BlockSpec((MERGE_BLK, SEQ, CTX_DIM), lambda i: (i, 0, 0)),
            pl.BlockSpec((MERGE_BLK, SEQ, CTX_DIM), lambda i: (i, 0, 0)),
        ],
        out_specs=pl.BlockSpec((NUM_CORES * MERGE_BLK, SEQ, CTX_DIM),
                               lambda i: (i, 0, 0)),
        out_shape=jax.ShapeDtypeStruct((B, SEQ, CTX_DIM), jnp.float32),
    )(out0, out1)
    return merged


def kernel(text, ctx, token_embedding):
    return _prompt_gather(text, ctx, token_embedding)


# speedup vs baseline: 1.5319x; 1.4897x over previous
import dataclasses

import jax
import jax.numpy as jnp
from jax.experimental import pallas as pl
from jax.experimental.pallas import tpu as pltpu
from jax.experimental.pallas import tpu_sc as plsc

B = 1024
SEQ = 77
CTX_DIM = 512
N_CTX = 4
KEEP = SEQ - N_CTX

NUM_CORES = 2
NUM_SUBCORES = 16
NUM_WORKERS = NUM_CORES * NUM_SUBCORES
BATCH_PER_W = B // NUM_WORKERS
NBUF = 3


def _sc_body(txt_hbm, ctx_hbm, table_hbm, out_hbm,
             idx_v, ctx_v, gbuf0, gbuf1, gbuf2, sidx, cidx,
             gsem0, gsem1, gsem2, wsem0, wsem1, wsem2, csem):
    wid = jax.lax.axis_index("s") * NUM_CORES + jax.lax.axis_index("c")
    base = wid * BATCH_PER_W

    gbufs = (gbuf0, gbuf1, gbuf2)
    gsems = (gsem0, gsem1, gsem2)
    wsems = (wsem0, wsem1, wsem2)

    pltpu.sync_copy(txt_hbm.at[pl.ds(base, BATCH_PER_W)], idx_v)
    pltpu.sync_copy(ctx_hbm, ctx_v)

    lane = jax.lax.iota(jnp.int32, 16)
    for c in range(KEEP // 16):
        sidx.at[pl.ds(16 * c, 16)][...] = lane + (N_CTX + 16 * c)
    tail = (KEEP // 16) * 16
    plsc.store_scatter(sidx, [lane + tail], lane + (N_CTX + tail),
                       mask=lane < (KEEP - tail))
    plsc.store_scatter(cidx, [lane], lane, mask=lane < N_CTX)

    def gather_copy(j, b):
        return pltpu.make_async_copy(
            table_hbm.at[idx_v.at[j, pl.ds(0, KEEP)]],
            gbufs[b],
            gsems[b])

    def scatter_copy(j, b):
        return pltpu.make_async_copy(
            gbufs[b],
            out_hbm.at[base + j].at[sidx],
            wsems[b])

    def ctx_copy(j):
        return pltpu.make_async_copy(
            ctx_v,
            out_hbm.at[base + j].at[cidx],
            csem)

    for b in range(NBUF):
        gather_copy(b, b).start()

    def step(j, b, refire):
        ctx_copy(j).start()
        gather_copy(j, b).wait()
        scatter_copy(j, b).start()
        scatter_copy(j, b).wait()
        if refire:
            gather_copy(j + NBUF, b).start()
        ctx_copy(j).wait()

    FULL = (BATCH_PER_W - NBUF) // NBUF

    @pl.loop(0, FULL)
    def _(g):
        for b in range(NBUF):
            step(NBUF * g + b, b, refire=True)

    for j in range(NBUF * FULL, BATCH_PER_W):
        step(j, j % NBUF, refire=(j + NBUF < BATCH_PER_W))


@jax.jit
def _prompt_gather(text, ctx, table):
    cp = pltpu.CompilerParams()
    if "needs_layout_passes" in pltpu.CompilerParams.__dataclass_fields__:
        cp = dataclasses.replace(cp, needs_layout_passes=False)
    grid_kernel = pl.kernel(
        _sc_body,
        compiler_params=cp,
        out_type=jax.ShapeDtypeStruct((B, SEQ, CTX_DIM), jnp.float32),
        mesh=plsc.VectorSubcoreMesh(core_axis_name="c", subcore_axis_name="s"),
        scratch_types=[
            pltpu.VMEM((BATCH_PER_W, SEQ), jnp.int32),
            pltpu.VMEM((N_CTX, CTX_DIM), jnp.float32),
            pltpu.VMEM((KEEP, CTX_DIM), jnp.float32),
            pltpu.VMEM((KEEP, CTX_DIM), jnp.float32),
            pltpu.VMEM((KEEP, CTX_DIM), jnp.float32),
            pltpu.VMEM((KEEP,), jnp.int32),
            pltpu.VMEM((N_CTX,), jnp.int32),
            pltpu.SemaphoreType.DMA,
            pltpu.SemaphoreType.DMA,
            pltpu.SemaphoreType.DMA,
            pltpu.SemaphoreType.DMA,
            pltpu.SemaphoreType.DMA,
            pltpu.SemaphoreType.DMA,
            pltpu.SemaphoreType.DMA,
        ],
    )
    return grid_kernel(text, ctx, table)


def kernel(text, ctx, token_embedding):
    return _prompt_gather(text, ctx, token_embedding)


# speedup vs baseline: 1.5398x; 1.0052x over previous
import dataclasses

import jax
import jax.numpy as jnp
from jax.experimental import pallas as pl
from jax.experimental.pallas import tpu as pltpu
from jax.experimental.pallas import tpu_sc as plsc

B = 1024
SEQ = 77
CTX_DIM = 512
N_CTX = 4
KEEP = SEQ - N_CTX

NUM_CORES = 2
NUM_SUBCORES = 16
NUM_WORKERS = NUM_CORES * NUM_SUBCORES
BATCH_PER_W = B // NUM_WORKERS
NBUF = 2


def _sc_body(txt_hbm, ctx_hbm, table_hbm, out_hbm,
             idx_v, ctx_v, gbuf0, gbuf1, sidx, cidx,
             gsem0, gsem1, wsem0, wsem1, csem):
    wid = jax.lax.axis_index("s") * NUM_CORES + jax.lax.axis_index("c")
    base = wid * BATCH_PER_W

    gbufs = (gbuf0, gbuf1)
    gsems = (gsem0, gsem1)
    wsems = (wsem0, wsem1)

    pltpu.sync_copy(txt_hbm.at[pl.ds(base, BATCH_PER_W)], idx_v)
    pltpu.sync_copy(ctx_hbm, ctx_v)

    lane = jax.lax.iota(jnp.int32, 16)
    for c in range(KEEP // 16):
        sidx.at[pl.ds(16 * c, 16)][...] = lane + (N_CTX + 16 * c)
    tail = (KEEP // 16) * 16
    plsc.store_scatter(sidx, [lane + tail], lane + (N_CTX + tail),
                       mask=lane < (KEEP - tail))
    plsc.store_scatter(cidx, [lane], lane, mask=lane < N_CTX)

    def gather_copy(j, b):
        return pltpu.make_async_copy(
            table_hbm.at[idx_v.at[j, pl.ds(0, KEEP)]],
            gbufs[b],
            gsems[b])

    def scatter_copy(j, b):
        return pltpu.make_async_copy(
            gbufs[b],
            out_hbm.at[base + j].at[sidx],
            wsems[b])

    def ctx_copy(j):
        return pltpu.make_async_copy(
            ctx_v,
            out_hbm.at[base + j].at[cidx],
            csem)

    gather_copy(0, 0).start()
    gather_copy(1, 1).start()

    def step(j, b, refire):
        ctx_copy(j).start()
        gather_copy(j, b).wait()
        scatter_copy(j, b).start()
        scatter_copy(j, b).wait()
        if refire:
            gather_copy(j + NBUF, b).start()
        ctx_copy(j).wait()

    @pl.loop(0, BATCH_PER_W // NBUF - 1)
    def _(g):
        for b in range(NBUF):
            step(NBUF * g + b, b, refire=True)

    for b in range(NBUF):
        step(BATCH_PER_W - NBUF + b, b, refire=False)


@jax.jit
def _prompt_gather(text, ctx, table):
    cp = pltpu.CompilerParams()
    if "needs_layout_passes" in pltpu.CompilerParams.__dataclass_fields__:
        cp = dataclasses.replace(cp, needs_layout_passes=False)
    grid_kernel = pl.kernel(
        _sc_body,
        compiler_params=cp,
        out_type=jax.ShapeDtypeStruct((B, SEQ, CTX_DIM), jnp.float32),
        mesh=plsc.VectorSubcoreMesh(core_axis_name="c", subcore_axis_name="s"),
        scratch_types=[
            pltpu.VMEM((BATCH_PER_W, SEQ), jnp.int32),
            pltpu.VMEM((N_CTX, CTX_DIM), jnp.float32),
            pltpu.VMEM((KEEP, CTX_DIM), jnp.float32),
            pltpu.VMEM((KEEP, CTX_DIM), jnp.float32),
            pltpu.VMEM((KEEP,), jnp.int32),
            pltpu.VMEM((N_CTX,), jnp.int32),
            pltpu.SemaphoreType.DMA,
            pltpu.SemaphoreType.DMA,
            pltpu.SemaphoreType.DMA,
            pltpu.SemaphoreType.DMA,
            pltpu.SemaphoreType.DMA,
        ],
    )
    return grid_kernel(text, ctx, table)


def kernel(text, ctx, token_embedding):
    return _prompt_gather(text, ctx, token_embedding)
